# baseline (speedup 1.0000x reference)
import functools

import jax
import jax.numpy as jnp
from jax import lax
from jax.experimental import pallas as pl
from jax.experimental.pallas import tpu as pltpu
from jax.experimental.pallas import tpu_sc as plsc

BATCH = 4096
EMBED_DIM = 64
TAG_LEN = 50
CAT_LEN = 20
LANES = 16
NGRP = EMBED_DIM // LANES
def _make_phase(bag_len, chunk_bags, nw, bags_w, NSLOT):
    nchunks = bags_w // chunk_bags
    rows = chunk_bags * bag_len
    mesh = plsc.VectorSubcoreMesh(core_axis_name="c", subcore_axis_name="s")
    out_sds = jax.ShapeDtypeStruct((BATCH * EMBED_DIM,), jnp.float32)

    @functools.partial(
        pl.kernel,
        mesh=mesh,
        out_type=out_sds,
        compiler_params=pltpu.CompilerParams(use_tc_tiling_on_sc=False),
        scratch_types=[
            pltpu.VMEM((bags_w * bag_len,), jnp.int32),
            pltpu.VMEM((NSLOT, rows, EMBED_DIM), jnp.float32),
            pltpu.VMEM((bags_w * EMBED_DIM,), jnp.float32),
        ] + [pltpu.SemaphoreType.DMA] * NSLOT,
    )
    def enc(idx_hbm, tab_hbm, out_hbm, idx_v, rows_v, out_v, *sems):
        ncores = 2
        wid = lax.axis_index("s") * ncores + lax.axis_index("c")
        nidx = bags_w * bag_len
        pltpu.sync_copy(idx_hbm.at[pl.ds(wid * nidx, nidx)], idx_v)

        def start(j, s):
            idx = idx_v.at[pl.ds(j * rows, rows)]
            pltpu.async_copy(tab_hbm.at[idx], rows_v.at[s], sems[s])

        def wait(s):
            idx = idx_v.at[pl.ds(0, rows)]
            pltpu.make_async_copy(tab_hbm.at[idx], rows_v.at[s],
                                  sems[s]).wait()

        def accumulate(j, s):
            rv = rows_v.at[s]

            unroll = 5

            def bag_body(bb, _):
                base = bb * bag_len
                accs = [rv[base, pl.ds(g * LANES, LANES)] for g in range(NGRP)]
                for l in range(1, unroll):
                    for g in range(NGRP):
                        accs[g] = accs[g] + rv[base + l,
                                               pl.ds(g * LANES, LANES)]

                def blk(t, accs):
                    row = base + t * unroll
                    for l in range(unroll):
                        accs = tuple(accs[g] + rv[row + l,
                                                  pl.ds(g * LANES, LANES)]
                                     for g in range(NGRP))
                    return accs

                accs = lax.fori_loop(1, bag_len // unroll, blk, tuple(accs))
                out_base = (j * chunk_bags + bb) * EMBED_DIM
                for g in range(NGRP):
                    out_v[pl.ds(out_base + g * LANES, LANES)] = accs[g]
                return _

            lax.fori_loop(0, chunk_bags, bag_body, None)

        for s in range(NSLOT - 1):
            start(s, s)

        def outer(jj, _):
            for s in range(NSLOT):
                j = jj * NSLOT + s
                wait(s)
                nxt = j + NSLOT - 1
                pl.when(nxt < nchunks)(
                    lambda: start(nxt, (s + NSLOT - 1) % NSLOT))
                accumulate(j, s)
            return _

        lax.fori_loop(0, nchunks // NSLOT, outer, None)
        nout = bags_w * EMBED_DIM
        pltpu.sync_copy(out_v, out_hbm.at[pl.ds(wid * nout, nout)])

    return enc


def kernel(tags, categories, tag_table, cat_table):
    info = plsc.get_sparse_core_info()
    nw = info.num_cores * info.num_subcores
    bags_w = BATCH // nw

    ttab = jnp.pad(tag_table, ((0, 0), (0, EMBED_DIM))).reshape(-1, EMBED_DIM)
    ctab = jnp.pad(cat_table, ((0, 0), (0, EMBED_DIM))).reshape(-1, EMBED_DIM)
    out_c = _make_phase(CAT_LEN, 8, nw, bags_w, 4)(
        categories.reshape(-1) * 2, ctab)
    out_t = _make_phase(TAG_LEN, 4, nw, bags_w, 8)(
        tags.reshape(-1) * 2, ttab)
    return (out_t.reshape(BATCH, EMBED_DIM), out_c.reshape(BATCH, EMBED_DIM))


# speedup vs baseline: 1.0033x; 1.0033x over previous
import functools

import jax
import jax.numpy as jnp
from jax import lax
from jax.experimental import pallas as pl
from jax.experimental.pallas import tpu as pltpu
from jax.experimental.pallas import tpu_sc as plsc

BATCH = 4096
EMBED_DIM = 64
TAG_LEN = 50
CAT_LEN = 20
LANES = 16
NGRP = EMBED_DIM // LANES
def _make_phase(bag_len, chunk_bags, nw, bags_w, NSLOT):
    nchunks = bags_w // chunk_bags
    rows = chunk_bags * bag_len
    mesh = plsc.VectorSubcoreMesh(core_axis_name="c", subcore_axis_name="s")
    out_sds = jax.ShapeDtypeStruct((BATCH * EMBED_DIM,), jnp.float32)

    @functools.partial(
        pl.kernel,
        mesh=mesh,
        out_type=out_sds,
        compiler_params=pltpu.CompilerParams(use_tc_tiling_on_sc=False),
        scratch_types=[
            pltpu.VMEM((bags_w * bag_len,), jnp.int32),
            pltpu.VMEM((NSLOT, rows, EMBED_DIM), jnp.float32),
            pltpu.VMEM((bags_w * EMBED_DIM,), jnp.float32),
        ] + [pltpu.SemaphoreType.DMA] * NSLOT,
    )
    def enc(idx_hbm, tab_hbm, out_hbm, idx_v, rows_v, out_v, *sems):
        ncores = 2
        wid = lax.axis_index("s") * ncores + lax.axis_index("c")
        nidx = bags_w * bag_len
        pltpu.sync_copy(idx_hbm.at[pl.ds(wid * nidx, nidx)], idx_v)

        def start(j, s):
            idx = idx_v.at[pl.ds(j * rows, rows)]
            pltpu.async_copy(tab_hbm.at[idx], rows_v.at[s], sems[s])

        def wait(s):
            idx = idx_v.at[pl.ds(0, rows)]
            pltpu.make_async_copy(tab_hbm.at[idx], rows_v.at[s],
                                  sems[s]).wait()

        def accumulate(j, s):
            rv = rows_v.at[s]

            unroll = 5

            def bag_body(bb, _):
                base = bb * bag_len
                accs = [rv[base, pl.ds(g * LANES, LANES)] for g in range(NGRP)]
                for l in range(1, unroll):
                    for g in range(NGRP):
                        accs[g] = accs[g] + rv[base + l,
                                               pl.ds(g * LANES, LANES)]

                def blk(t, accs):
                    row = base + t * unroll
                    for l in range(unroll):
                        accs = tuple(accs[g] + rv[row + l,
                                                  pl.ds(g * LANES, LANES)]
                                     for g in range(NGRP))
                    return accs

                accs = lax.fori_loop(1, bag_len // unroll, blk, tuple(accs))
                out_base = (j * chunk_bags + bb) * EMBED_DIM
                for g in range(NGRP):
                    out_v[pl.ds(out_base + g * LANES, LANES)] = accs[g]
                return _

            lax.fori_loop(0, chunk_bags, bag_body, None)

        for s in range(NSLOT - 1):
            start(s, s)

        def outer(jj, _):
            for s in range(NSLOT):
                j = jj * NSLOT + s
                wait(s)
                nxt = j + NSLOT - 1
                pl.when(nxt < nchunks)(
                    lambda: start(nxt, (s + NSLOT - 1) % NSLOT))
                accumulate(j, s)
            return _

        lax.fori_loop(0, nchunks // NSLOT, outer, None)
        nout = bags_w * EMBED_DIM
        pltpu.sync_copy(out_v, out_hbm.at[pl.ds(wid * nout, nout)])

    return enc


def kernel(tags, categories, tag_table, cat_table):
    info = plsc.get_sparse_core_info()
    nw = info.num_cores * info.num_subcores
    bags_w = BATCH // nw

    ttab = jnp.pad(tag_table, ((0, 0), (0, EMBED_DIM))).reshape(-1, EMBED_DIM)
    ctab = jnp.pad(cat_table, ((0, 0), (0, EMBED_DIM))).reshape(-1, EMBED_DIM)
    out_c = _make_phase(CAT_LEN, 8, nw, bags_w, 4)(
        categories.reshape(-1) * 2, ctab)
    out_t = _make_phase(TAG_LEN, 8, nw, bags_w, 4)(
        tags.reshape(-1) * 2, ttab)
    return (out_t.reshape(BATCH, EMBED_DIM), out_c.reshape(BATCH, EMBED_DIM))
